# baseline (speedup 1.0000x reference)
import functools

import jax
import jax.numpy as jnp
from jax import lax
from jax.experimental import pallas as pl
from jax.experimental.pallas import tpu as pltpu
from jax.experimental.pallas import tpu_sc as plsc

NBUF = 4
PAIR = 2
CHUNK = 128


def _make_lookup(tot, vocab, dim, num_workers):
    assert tot % (num_workers * CHUNK) == 0
    rows_per_worker = tot // num_workers
    chunks = rows_per_worker // CHUNK
    pairs = chunks // PAIR
    slots = NBUF // PAIR
    rounds = pairs // slots
    assert chunks % (NBUF * PAIR) == 0 or pairs % slots == 0

    mesh = plsc.VectorSubcoreMesh(core_axis_name="c", subcore_axis_name="s")

    @functools.partial(
        pl.kernel,
        mesh=mesh,
        out_type=jax.ShapeDtypeStruct((tot, dim), jnp.float32),
        scratch_types=(
            [
                pltpu.VMEM_SHARED((vocab, dim), jnp.float32),
                pltpu.VMEM((chunks, CHUNK), jnp.int32),
            ]
            + [pltpu.VMEM((PAIR * CHUNK, dim), jnp.float32) for _ in range(NBUF // PAIR)]
            + [pltpu.SemaphoreType.DMA for _ in range(NBUF // PAIR)]
            + [pltpu.SemaphoreType.DMA for _ in range(NBUF // PAIR)]
        ),
    )
    def lookup(idx_hbm, table_hbm, out_hbm, table_sp, idx_v, *scratch):
        nslots = NBUF // PAIR
        bufs = scratch[:nslots]
        gsems = scratch[nslots : 2 * nslots]
        ssems = scratch[2 * nslots :]
        nc = lax.axis_size("c")
        sid = lax.axis_index("s")
        wid = sid * nc + lax.axis_index("c")

        @pl.when(sid == 0)
        def _():
            pltpu.sync_copy(table_hbm, table_sp)

        pltpu.sync_copy(idx_hbm.at[pl.ds(wid * chunks, chunks)], idx_v)
        plsc.subcore_barrier()
        row_base = wid * rows_per_worker

        def gathers(b, p):
            for q in range(PAIR):
                pltpu.async_copy(
                    table_sp.at[idx_v.at[p * PAIR + q]],
                    bufs[b].at[pl.ds(q * CHUNK, CHUNK)],
                    gsems[b],
                )

        def wait_gathers(b):
            pltpu.make_async_copy(table_sp.at[idx_v.at[0]], bufs[b], gsems[b]).wait()

        def out_slice(p):
            return out_hbm.at[pl.ds(row_base + p * PAIR * CHUNK, PAIR * CHUNK)]

        for b in range(nslots):
            gathers(b, b)

        def round_step(r, carry):
            for b in range(nslots):
                p = r * nslots + b
                wait_gathers(b)
                pltpu.async_copy(bufs[b], out_slice(p), ssems[b])

                @pl.when(p + nslots < pairs)
                def _():
                    pltpu.make_async_copy(bufs[b], out_slice(p), ssems[b]).wait()
                    gathers(b, p + nslots)

            return carry

        lax.fori_loop(0, rounds, round_step, 0)
        for b in range(nslots):
            pltpu.make_async_copy(bufs[b], out_slice(pairs - nslots + b), ssems[b]).wait()

    return lookup


def kernel(char_sequences, char_emb_table):
    batch, word_len = char_sequences.shape
    vocab, dim = char_emb_table.shape
    tot = batch * word_len
    idx2d = char_sequences.T.reshape(tot // CHUNK, CHUNK)
    info = plsc.get_sparse_core_info()
    num_workers = info.num_cores * info.num_subcores
    flat = _make_lookup(tot, vocab, dim, num_workers)(idx2d, char_emb_table)
    return jnp.transpose(flat.reshape(word_len, batch, dim), (1, 0, 2))


# speedup vs baseline: 1.0368x; 1.0368x over previous
import functools

import jax
import jax.numpy as jnp
from jax import lax
from jax.experimental import pallas as pl
from jax.experimental.pallas import tpu as pltpu
from jax.experimental.pallas import tpu_sc as plsc

NBUF = 5
CHUNK = 128


def _make_lookup(tot, vocab, dim, num_workers):
    assert tot % (num_workers * CHUNK) == 0
    rows_per_worker = tot // num_workers
    chunks = rows_per_worker // CHUNK
    rounds = chunks // NBUF
    assert chunks % NBUF == 0

    mesh = plsc.VectorSubcoreMesh(core_axis_name="c", subcore_axis_name="s")

    @functools.partial(
        pl.kernel,
        mesh=mesh,
        out_type=jax.ShapeDtypeStruct((tot, dim), jnp.float32),
        scratch_types=(
            [
                pltpu.VMEM_SHARED((vocab, dim), jnp.float32),
                pltpu.VMEM((chunks, CHUNK), jnp.int32),
            ]
            + [pltpu.VMEM((CHUNK, dim), jnp.float32) for _ in range(NBUF)]
            + [pltpu.SemaphoreType.DMA for _ in range(2 * NBUF)]
        ),
    )
    def lookup(idx_hbm, table_hbm, out_hbm, table_sp, idx_v, *scratch):
        bufs = scratch[:NBUF]
        gsems = scratch[NBUF : 2 * NBUF]
        ssems = scratch[2 * NBUF :]
        nc = lax.axis_size("c")
        sid = lax.axis_index("s")
        wid = sid * nc + lax.axis_index("c")

        @pl.when(sid == 0)
        def _():
            pltpu.sync_copy(table_hbm, table_sp)

        pltpu.sync_copy(idx_hbm.at[pl.ds(wid * chunks, chunks)], idx_v)
        plsc.subcore_barrier()
        row_base = wid * rows_per_worker

        def gather(b, j):
            pltpu.async_copy(table_sp.at[idx_v.at[j]], bufs[b], gsems[b])

        def out_slice(j):
            return out_hbm.at[pl.ds(row_base + j * CHUNK, CHUNK)]

        for b in range(NBUF):
            gather(b, b)

        def round_step(r, carry):
            for b in range(NBUF):
                j = r * NBUF + b
                pltpu.make_async_copy(table_sp.at[idx_v.at[j]], bufs[b], gsems[b]).wait()
                pltpu.async_copy(bufs[b], out_slice(j), ssems[b])

                @pl.when(j + NBUF < chunks)
                def _():
                    pltpu.make_async_copy(bufs[b], out_slice(j), ssems[b]).wait()
                    gather(b, j + NBUF)

            return carry

        lax.fori_loop(0, rounds, round_step, 0)
        for b in range(NBUF):
            pltpu.make_async_copy(bufs[b], out_slice(chunks - NBUF + b), ssems[b]).wait()

    return lookup


def kernel(char_sequences, char_emb_table):
    batch, word_len = char_sequences.shape
    vocab, dim = char_emb_table.shape
    tot = batch * word_len
    idx2d = char_sequences.T.reshape(tot // CHUNK, CHUNK)
    info = plsc.get_sparse_core_info()
    num_workers = info.num_cores * info.num_subcores
    flat = _make_lookup(tot, vocab, dim, num_workers)(idx2d, char_emb_table)
    return jnp.transpose(flat.reshape(word_len, batch, dim), (1, 0, 2))
